# Initial kernel scaffold; baseline (speedup 1.0000x reference)
import jax
import jax.numpy as jnp
from jax.experimental import pallas as pl


def kernel(x, edge_index, eps, W1, b1, g1, be1, W2, b2, g2, be2):
    raise NotImplementedError("write your pallas kernel here")


# speedup vs baseline: 3.1039x; 3.1039x over previous
import functools

import jax
import jax.numpy as jnp
from jax import lax
from jax.experimental import pallas as pl
from jax.experimental.pallas import tpu as pltpu
from jax.experimental.pallas import tpu_sc as plsc

_N, _D = 10000, 128
_NC, _NS = 2, 16
_NW = _NC * _NS
_CHUNK = 128
_CPT = 80
_EPAD = _NW * _CPT * _CHUNK
_RPT = 624
_TAIL = _N - _NS * _RPT
_AGG_ROWS = _N + 8
_BN_EPS = 1e-5


def _sc_partials(x, row2d, col2d):
    mesh = plsc.VectorSubcoreMesh(core_axis_name="c", subcore_axis_name="s")

    @functools.partial(
        pl.kernel,
        out_type=jax.ShapeDtypeStruct((_NC, _N, _D), jnp.float32),
        mesh=mesh,
        scratch_types=[
            pltpu.VMEM((_CPT // 2, _CHUNK), jnp.int32),
            pltpu.VMEM((_CPT // 2, _CHUNK), jnp.int32),
            pltpu.VMEM((_CHUNK, _D), jnp.float32),
            pltpu.VMEM((_CHUNK, _D), jnp.float32),
            pltpu.VMEM_SHARED((_AGG_ROWS, _D), jnp.float32),
            pltpu.SemaphoreType.DMA,
            pltpu.SemaphoreType.DMA,
        ],
    )
    def k(x_hbm, row_hbm, col_hbm, out_hbm, idx_r, idx_c, g0, g1, agg, s0, s1):
        c = lax.axis_index("c")
        s = lax.axis_index("s")
        wid = c * _NS + s
        pltpu.sync_copy(x_hbm.at[pl.ds(s * _RPT, _RPT)],
                        agg.at[pl.ds(s * _RPT, _RPT)])

        @pl.when(s < 2)
        def _():
            base = _NS * _RPT + s * 8
            pltpu.sync_copy(x_hbm.at[pl.ds(base, 8)], agg.at[pl.ds(base, 8)])

        plsc.subcore_barrier()

        def pair(i, carry):
            j0 = i * 2
            j1 = j0 + 1
            cp0 = pltpu.async_copy(x_hbm.at[idx_c.at[j0]], g0, s0)
            cp1 = pltpu.async_copy(x_hbm.at[idx_c.at[j1]], g1, s1)
            cp0.wait()
            pltpu.sync_copy(g0, agg.at[idx_r.at[j0]], add=True)
            cp1.wait()
            pltpu.sync_copy(g1, agg.at[idx_r.at[j1]], add=True)
            return carry

        for grp in range(2):
            base = wid * _CPT + grp * (_CPT // 2)
            pltpu.sync_copy(row_hbm.at[pl.ds(base, _CPT // 2)], idx_r)
            pltpu.sync_copy(col_hbm.at[pl.ds(base, _CPT // 2)], idx_c)
            lax.fori_loop(0, _CPT // 4, pair, 0)
        plsc.subcore_barrier()
        pltpu.sync_copy(agg.at[pl.ds(s * _RPT, _RPT)],
                        out_hbm.at[c].at[pl.ds(s * _RPT, _RPT)])

        @pl.when(s < 2)
        def _():
            base = _NS * _RPT + s * 8
            pltpu.sync_copy(agg.at[pl.ds(base, 8)],
                            out_hbm.at[c].at[pl.ds(base, 8)])

    return k(x, row2d, col2d)


def _tc_finish(x, parts, eps11, W1, b1, g1, be1, W2, b2, g2, be2):
    def body(x_ref, p_ref, eps_ref, w1_ref, b1_ref, g1_ref, be1_ref,
             w2_ref, b2_ref, g2_ref, be2_ref, o_ref):
        eps = eps_ref[0, 0]
        h = p_ref[0] + p_ref[1] + (eps - 1.0) * x_ref[...]
        h = jnp.dot(h, w1_ref[...], preferred_element_type=jnp.float32) + b1_ref[...]
        m = jnp.mean(h, axis=0, keepdims=True)
        v = jnp.mean((h - m) * (h - m), axis=0, keepdims=True)
        h = (h - m) * lax.rsqrt(v + _BN_EPS) * g1_ref[...] + be1_ref[...]
        h = jnp.maximum(h, 0.0)
        h = jnp.dot(h, w2_ref[...], preferred_element_type=jnp.float32) + b2_ref[...]
        m = jnp.mean(h, axis=0, keepdims=True)
        v = jnp.mean((h - m) * (h - m), axis=0, keepdims=True)
        h = (h - m) * lax.rsqrt(v + _BN_EPS) * g2_ref[...] + be2_ref[...]
        o_ref[...] = jnp.maximum(h, 0.0)

    return pl.pallas_call(
        body,
        out_shape=jax.ShapeDtypeStruct((_N, _D), jnp.float32),
    )(x, parts, eps11, W1, b1, g1, be1, W2, b2, g2, be2)


def kernel(x, edge_index, eps, W1, b1, g1, be1, W2, b2, g2, be2):
    e = edge_index.shape[1]
    pad = _EPAD - e
    row = jnp.concatenate(
        [edge_index[0], jnp.full((pad,), _N, jnp.int32)]).reshape(-1, _CHUNK)
    col = jnp.concatenate(
        [edge_index[1], jnp.zeros((pad,), jnp.int32)]).reshape(-1, _CHUNK)
    parts = _sc_partials(x, row, col)
    eps11 = jnp.reshape(eps, (1, 1)).astype(jnp.float32)
    return _tc_finish(
        x, parts, eps11,
        W1, b1.reshape(1, _D), g1.reshape(1, _D), be1.reshape(1, _D),
        W2, b2.reshape(1, _D), g2.reshape(1, _D), be2.reshape(1, _D))
